# Initial kernel scaffold; baseline (speedup 1.0000x reference)
import jax
import jax.numpy as jnp
from jax.experimental import pallas as pl


def kernel(x, labels, centers):
    raise NotImplementedError("write your pallas kernel here")


# speedup vs baseline: 1.0670x; 1.0670x over previous
import functools

import jax
import jax.numpy as jnp
from jax import lax
from jax.experimental import pallas as pl
from jax.experimental.pallas import tpu as pltpu
from jax.experimental.pallas import tpu_sc as plsc

_BATCH = 16384
_D = 128
_NC = 2
_NS = 16
_L = 16
_NW = _NC * _NS
_ROWS_PER_W = _BATCH // _NW
_CHUNK = 256
_N_CHUNKS = _ROWS_PER_W // _CHUNK


def _sc_partials(x, labels, centers):
    mesh = plsc.VectorSubcoreMesh(core_axis_name="c", subcore_axis_name="s")

    @functools.partial(
        pl.kernel,
        mesh=mesh,
        out_type=jax.ShapeDtypeStruct((_NW, _L), jnp.float32),
        scratch_types=[
            pltpu.VMEM((_CHUNK,), jnp.int32),
            pltpu.VMEM((_CHUNK, _D), jnp.float32),
            pltpu.VMEM((_CHUNK, _D), jnp.float32),
            pltpu.VMEM((_L,), jnp.float32),
            pltpu.SemaphoreType.DMA,
        ],
    )
    def body(x_hbm, labels_hbm, centers_hbm, out_hbm, idx_v, xbuf, cbuf,
             acc_v, sem):
        wid = lax.axis_index("s") * _NC + lax.axis_index("c")
        base = wid * _ROWS_PER_W
        acc = jnp.zeros((_L,), jnp.float32)
        for c in range(_N_CHUNKS):
            start = base + c * _CHUNK
            pltpu.sync_copy(labels_hbm.at[pl.ds(start, _CHUNK)], idx_v)
            gather = pltpu.async_copy(centers_hbm.at[idx_v], cbuf, sem)
            pltpu.sync_copy(x_hbm.at[pl.ds(start, _CHUNK), :], xbuf)
            gather.wait()

            def row_body(r, a):
                for j in range(_D // _L):
                    xv = xbuf[r, pl.ds(j * _L, _L)]
                    cv = cbuf[r, pl.ds(j * _L, _L)]
                    d = xv - cv
                    a = a + d * d
                return a

            acc = lax.fori_loop(0, _CHUNK, row_body, acc)
        acc_v[...] = acc
        pltpu.sync_copy(acc_v, out_hbm.at[wid])

    return body(x, labels, centers)


def _finish(partials):
    def body(p_ref, o_ref):
        o_ref[0, 0] = jnp.sum(p_ref[...]) * (1.0 / _BATCH)

    out = pl.pallas_call(
        body,
        out_shape=jax.ShapeDtypeStruct((1, 1), jnp.float32),
        out_specs=pl.BlockSpec(memory_space=pltpu.SMEM),
    )(partials)
    return out[0, 0]


def kernel(x, labels, centers):
    partials = _sc_partials(x, labels.astype(jnp.int32), centers)
    return _finish(partials)


# speedup vs baseline: 1.1127x; 1.0429x over previous
import functools

import jax
import jax.numpy as jnp
from jax import lax
from jax.experimental import pallas as pl
from jax.experimental.pallas import tpu as pltpu
from jax.experimental.pallas import tpu_sc as plsc

_BATCH = 16384
_D = 128
_NG = _D // 16
_NC = 2
_NS = 16
_L = 16
_NW = _NC * _NS
_ROWS_PER_W = _BATCH // _NW
_CHUNK = 128
_N_CHUNKS = _ROWS_PER_W // _CHUNK
_NBUF = 2


def _sc_partials(x, labels, centers):
    mesh = plsc.VectorSubcoreMesh(core_axis_name="c", subcore_axis_name="s")

    @functools.partial(
        pl.kernel,
        mesh=mesh,
        out_type=jax.ShapeDtypeStruct((_NW, _L), jnp.float32),
        scratch_types=[
            pltpu.VMEM((_N_CHUNKS, _CHUNK), jnp.int32),
            pltpu.VMEM((_NBUF, _CHUNK, _D), jnp.float32),
            pltpu.VMEM((_NBUF, _CHUNK, _D), jnp.float32),
            pltpu.VMEM((_L,), jnp.float32),
            pltpu.SemaphoreType.DMA,
            pltpu.SemaphoreType.DMA,
            pltpu.SemaphoreType.DMA,
            pltpu.SemaphoreType.DMA,
        ],
    )
    def body(x_hbm, labels_hbm, centers_hbm, out_hbm, idx_v, xbuf, cbuf,
             acc_v, gsem0, gsem1, xsem0, xsem1):
        gsems = (gsem0, gsem1)
        xsems = (xsem0, xsem1)
        wid = lax.axis_index("s") * _NC + lax.axis_index("c")
        base = wid * _ROWS_PER_W
        for c in range(_N_CHUNKS):
            pltpu.sync_copy(labels_hbm.at[pl.ds(base + c * _CHUNK, _CHUNK)],
                            idx_v.at[c])

        def start(c, b):
            g = pltpu.async_copy(centers_hbm.at[idx_v.at[c]], cbuf.at[b],
                                 gsems[b])
            xcp = pltpu.async_copy(
                x_hbm.at[pl.ds(base + c * _CHUNK, _CHUNK), :],
                xbuf.at[b], xsems[b])
            return g, xcp

        accs = [jnp.zeros((_L,), jnp.float32) for _ in range(_NG)]
        inflight = start(0, 0)
        for c in range(_N_CHUNKS):
            b = c % _NBUF
            cur = inflight
            if c + 1 < _N_CHUNKS:
                inflight = start(c + 1, (c + 1) % _NBUF)
            cur[0].wait()
            cur[1].wait()

            def row_body(r, a, _b=b):
                out = []
                for j in range(_NG):
                    xv = xbuf[_b, r, pl.ds(j * _L, _L)]
                    cv = cbuf[_b, r, pl.ds(j * _L, _L)]
                    d = xv - cv
                    out.append(a[j] + d * d)
                return tuple(out)

            accs = list(lax.fori_loop(0, _CHUNK, row_body, tuple(accs)))

        total = accs[0]
        for j in range(1, _NG):
            total = total + accs[j]
        acc_v[...] = total
        pltpu.sync_copy(acc_v, out_hbm.at[wid])

    return body(x, labels, centers)


def _finish(partials):
    def body(p_ref, o_ref):
        o_ref[0, 0] = jnp.sum(p_ref[...]) * (1.0 / _BATCH)

    out = pl.pallas_call(
        body,
        out_shape=jax.ShapeDtypeStruct((1, 1), jnp.float32),
        out_specs=pl.BlockSpec(memory_space=pltpu.SMEM),
    )(partials)
    return out[0, 0]


def kernel(x, labels, centers):
    partials = _sc_partials(x, labels.astype(jnp.int32), centers)
    return _finish(partials)


# speedup vs baseline: 1.1913x; 1.0706x over previous
import functools

import jax
import jax.numpy as jnp
from jax import lax
from jax.experimental import pallas as pl
from jax.experimental.pallas import tpu as pltpu
from jax.experimental.pallas import tpu_sc as plsc

_BATCH = 16384
_D = 128
_NG = _D // 16
_NC = 2
_NS = 16
_L = 16
_NW = _NC * _NS
_ROWS_PER_W = _BATCH // _NW
_CHUNK = 64
_N_CHUNKS = _ROWS_PER_W // _CHUNK
_NBUF = 4
_DEPTH = 3


def _sc_partials(x, labels, centers):
    mesh = plsc.VectorSubcoreMesh(core_axis_name="c", subcore_axis_name="s")

    @functools.partial(
        pl.kernel,
        mesh=mesh,
        out_type=jax.ShapeDtypeStruct((_NW, _L), jnp.float32),
        scratch_types=[
            pltpu.VMEM((_ROWS_PER_W,), jnp.int32),
            pltpu.VMEM((_NBUF, _CHUNK, _D), jnp.float32),
            pltpu.VMEM((_NBUF, _CHUNK, _D), jnp.float32),
            pltpu.VMEM((_L,), jnp.float32),
            pltpu.SemaphoreType.DMA,
            pltpu.SemaphoreType.DMA,
            pltpu.SemaphoreType.DMA,
            pltpu.SemaphoreType.DMA,
        ],
    )
    def body(x_hbm, labels_hbm, centers_hbm, out_hbm, idx_v, xbuf, cbuf,
             acc_v, sem0, sem1, sem2, sem3):
        sems = (sem0, sem1, sem2, sem3)
        wid = lax.axis_index("s") * _NC + lax.axis_index("c")
        base = wid * _ROWS_PER_W
        pltpu.sync_copy(labels_hbm.at[pl.ds(base, _ROWS_PER_W)], idx_v)

        def start(c, b):
            g = pltpu.async_copy(
                centers_hbm.at[idx_v.at[pl.ds(c * _CHUNK, _CHUNK)]],
                cbuf.at[b], sems[b])
            xcp = pltpu.async_copy(
                x_hbm.at[pl.ds(base + c * _CHUNK, _CHUNK), :],
                xbuf.at[b], sems[b])
            return g, xcp

        accs = [jnp.zeros((_L,), jnp.float32) for _ in range(_NG)]
        inflight = [start(c, c % _NBUF) for c in range(_DEPTH)]
        for c in range(_N_CHUNKS):
            b = c % _NBUF
            cur = inflight.pop(0)
            if c + _DEPTH < _N_CHUNKS:
                inflight.append(start(c + _DEPTH, (c + _DEPTH) % _NBUF))
            cur[0].wait()
            cur[1].wait()

            def row_body(r, a, _b=b):
                out = []
                for j in range(_NG):
                    xv = xbuf[_b, r, pl.ds(j * _L, _L)]
                    cv = cbuf[_b, r, pl.ds(j * _L, _L)]
                    d = xv - cv
                    out.append(a[j] + d * d)
                return tuple(out)

            accs = list(lax.fori_loop(0, _CHUNK, row_body, tuple(accs)))

        total = accs[0]
        for j in range(1, _NG):
            total = total + accs[j]
        acc_v[...] = total
        pltpu.sync_copy(acc_v, out_hbm.at[wid])

    return body(x, labels, centers)


def _finish(partials):
    def body(p_ref, o_ref):
        o_ref[0, 0] = jnp.sum(p_ref[...]) * (1.0 / _BATCH)

    out = pl.pallas_call(
        body,
        out_shape=jax.ShapeDtypeStruct((1, 1), jnp.float32),
        out_specs=pl.BlockSpec(memory_space=pltpu.SMEM),
    )(partials)
    return out[0, 0]


def kernel(x, labels, centers):
    partials = _sc_partials(x, labels.astype(jnp.int32), centers)
    return _finish(partials)
